# baseline (speedup 1.0000x reference)
import functools

import jax
import jax.numpy as jnp
from jax import lax
from jax.experimental import pallas as pl
from jax.experimental.pallas import tpu as pltpu
from jax.experimental.pallas import tpu_sc as plsc

_BETA = 0.05
_MARGIN = 1.0
_LANES = 16

_KS, _KSN, _KQ, _KQN = 5, 5, 10, 10
_RPB = 2 * (_KS + _KSN + _KQ + _KQN)

_NC, _NS = 2, 16
_NW = _NC * _NS
_GW = 120


def _rsqrt(x):
    i = lax.bitcast_convert_type(x, jnp.int32)
    i = jnp.int32(0x5F3759DF) - lax.shift_right_arithmetic(i, jnp.int32(1))
    y = lax.bitcast_convert_type(i, jnp.float32)
    for _ in range(3):
        y = y * (1.5 - 0.5 * x * y * y)
    return y


def _sc_body(nbw, d, gscale, emb, idx, out_p, out_n, idx_v, rows_v,
             p_stage, n_stage, sem):
    nc = d // _LANES
    wid = lax.axis_index("s") * _NC + lax.axis_index("c")
    n_gath = nbw * _RPB // _GW

    pltpu.sync_copy(idx.at[pl.ds(wid * n_gath, n_gath)], idx_v)
    copies = [
        pltpu.async_copy(emb.at[idx_v.at[j]],
                         rows_v.at[pl.ds(j * _GW, _GW)], sem)
        for j in range(n_gath)
    ]
    for cp in copies:
        cp.wait()

    def row(i, c):
        return rows_v[i, pl.ds(c * _LANES, _LANES)]

    def one_task(lb, carry):
        bo = lb * _RPB
        r = []
        for c in range(nc):
            acc = row(bo + _KS, c) - row(bo, c)
            for k in range(1, _KS):
                acc = acc + (row(bo + _KS + k, c) - row(bo + k, c))
            r.append(acc * (1.0 / _KS))

        g = None
        for j in range(_KS):
            dp = [row(bo + j, c) + r[c] - row(bo + _KS + j, c)
                  for c in range(nc)]
            sp = jnp.sum(sum(v * v for v in dp))
            dn = [row(bo + 2 * _KS + j, c) + r[c]
                  - row(bo + 3 * _KS + j, c) for c in range(nc)]
            sn = jnp.sum(sum(v * v for v in dn))
            rsp = _rsqrt(sp)
            rsn = _rsqrt(sn)
            ind = (_MARGIN - sn * rsn + sp * rsp) > 0.0
            cp_ = jnp.where(ind, rsp, 0.0)
            cn_ = jnp.where(ind, rsn, 0.0)
            contrib = [cp_ * dp[c] - cn_ * dn[c] for c in range(nc)]
            g = contrib if g is None else [g[c] + contrib[c]
                                           for c in range(nc)]

        relq = [r[c] - gscale * g[c] for c in range(nc)]

        lane = lax.broadcasted_iota(jnp.int32, (_LANES,), 0)
        qb = bo + 4 * _KS
        pacc = jnp.zeros((_LANES,), jnp.float32)
        for q in range(_KQ):
            dq = [row(qb + q, c) + relq[c] - row(qb + _KQ + q, c)
                  for c in range(nc)]
            s = jnp.sum(sum(v * v for v in dq))
            pacc = jnp.where(lane == q, -(s * _rsqrt(s)), pacc)
        p_stage[pl.ds(lb * _KQ, _LANES)] = pacc
        nb = qb + 2 * _KQ
        nacc = jnp.zeros((_LANES,), jnp.float32)
        for q in range(_KQN):
            dq = [row(nb + q, c) + relq[c] - row(nb + _KQN + q, c)
                  for c in range(nc)]
            s = jnp.sum(sum(v * v for v in dq))
            nacc = jnp.where(lane == q, -(s * _rsqrt(s)), nacc)
        n_stage[pl.ds(lb * _KQN, _LANES)] = nacc
        return carry

    lax.fori_loop(0, nbw, one_task, 0)

    pltpu.sync_copy(p_stage.at[pl.ds(0, nbw * _KQ)],
                    out_p.at[pl.ds(wid * nbw * _KQ, nbw * _KQ)])
    pltpu.sync_copy(n_stage.at[pl.ds(0, nbw * _KQN)],
                    out_n.at[pl.ds(wid * nbw * _KQN, nbw * _KQN)])


def kernel(embeddings, support_idx, support_neg_idx, query_idx,
           query_neg_idx):
    b = support_idx.shape[0]
    d = embeddings.shape[1]
    nbw = b // _NW
    gscale = _BETA / (b * _KS)

    flat = jnp.concatenate([
        support_idx[:, :, 0], support_idx[:, :, 1],
        support_neg_idx[:, :, 0], support_neg_idx[:, :, 1],
        query_idx[:, :, 0], query_idx[:, :, 1],
        query_neg_idx[:, :, 0], query_neg_idx[:, :, 1],
    ], axis=1)
    idx2d = flat.reshape(b * _RPB // _GW, _GW)

    fn = pl.kernel(
        functools.partial(_sc_body, nbw, d, gscale),
        mesh=plsc.VectorSubcoreMesh(core_axis_name="c",
                                    subcore_axis_name="s"),
        out_type=(
            jax.ShapeDtypeStruct((b * _KQ,), jnp.float32),
            jax.ShapeDtypeStruct((b * _KQN,), jnp.float32),
        ),
        scratch_types=[
            pltpu.VMEM((nbw * _RPB // _GW, _GW), jnp.int32),
            pltpu.VMEM((nbw * _RPB, d), jnp.float32),
            pltpu.VMEM((nbw * _KQ + 8, ), jnp.float32),
            pltpu.VMEM((nbw * _KQN + 8, ), jnp.float32),
            pltpu.SemaphoreType.DMA,
        ],
        compiler_params=pltpu.CompilerParams(needs_layout_passes=False,
                                             use_tc_tiling_on_sc=False),
    )
    p_flat, n_flat = fn(embeddings, idx2d)
    return p_flat.reshape(b, _KQ), n_flat.reshape(b, _KQN)


# speedup vs baseline: 1.4509x; 1.4509x over previous
import functools

import jax
import jax.numpy as jnp
from jax import lax
from jax.experimental import pallas as pl
from jax.experimental.pallas import tpu as pltpu
from jax.experimental.pallas import tpu_sc as plsc

_BETA = 0.05
_MARGIN = 1.0
_LANES = 16

_KS, _KSN, _KQ, _KQN = 5, 5, 10, 10
_RPB = 2 * (_KS + _KSN + _KQ + _KQN)

_NC, _NS = 2, 16
_NW = _NC * _NS

_GSZ = 512
_CAP_LOC = 6144
_CAP_GRP = 128


def _rsqrt(x):
    i = lax.bitcast_convert_type(x, jnp.int32)
    i = jnp.int32(0x5F3759DF) - lax.shift_right_arithmetic(i, jnp.int32(1))
    y = lax.bitcast_convert_type(i, jnp.float32)
    for _ in range(3):
        y = y * (1.5 - 0.5 * x * y * y)
    return y


def _extract_body(n_inst, n_ent, emb_t, aux, elist, r_out,
                  el_v, loc_e, loc_i, ge, gi, slab_v, aux_v, stage_v, sem):
    t = lax.axis_index("s") * _NC + lax.axis_index("c")
    lane = lax.broadcasted_iota(jnp.int32, (_LANES,), 0)

    ch = 16
    chw = n_inst // ch

    def chunk_body(ci, off):
        pltpu.sync_copy(elist.at[pl.ds(ci * chw, chw)], el_v)

        def vloop(vi, off):
            for u in range(16):
                vbase = (vi * 16 + u) * 16
                ev = el_v[pl.ds(vbase, 16)]
                m = (lax.shift_right_logical(ev, 9) & 31) == t
                c = plsc.all_reduce_population_count(m)[0]
                plsc.store_compressed(loc_e.at[pl.ds(off, 16)], ev, mask=m)
                iv = ci * chw + vbase + lane
                plsc.store_compressed(loc_i.at[pl.ds(off, 16)], iv, mask=m)
                off = jnp.minimum(off + c, _CAP_LOC)
            return off

        return lax.fori_loop(0, chw // 256, vloop, off)

    cnt_loc = lax.fori_loop(0, ch, chunk_body, jnp.int32(0))

    def process_group(g, base, src, width):
        nv = (cnt_loc + 15) // 16

        def rescan(v, goff):
            ev = loc_e[pl.ds(v * 16, 16)]
            iv = loc_i[pl.ds(v * 16, 16)]
            valid = (v * 16 + lane) < cnt_loc
            m = valid & (lax.shift_right_logical(ev, 9) == g)
            c = plsc.all_reduce_population_count(m)[0]
            plsc.store_compressed(ge.at[pl.ds(goff, 16)], ev, mask=m)
            plsc.store_compressed(gi.at[pl.ds(goff, 16)], iv, mask=m)
            return jnp.minimum(goff + c, _CAP_GRP)

        cnt_g = lax.fori_loop(0, nv, rescan, jnp.int32(0))
        nk = (cnt_g + 15) // 16

        def extract(k, _):
            ev = ge[pl.ds(k * 16, 16)]
            for u in range(16):
                col = jnp.clip(ev[u] - base, 0, width - 1)
                colv = jnp.broadcast_to(col, (_LANES,))
                for c4 in range(4):
                    rowv = lane + c4 * 16
                    val = plsc.load_gather(src, [rowv, colv])
                    stage_v[k * 16 + u, pl.ds(c4 * 16, 16)] = val
            return 0

        lax.fori_loop(0, nk, extract, 0)

        def scat(k, _):
            iv = gi[pl.ds(k * 16, 16)]
            valid = (k * 16 + lane) < cnt_g
            ivp = jnp.where(valid, iv, n_inst + lane)
            pltpu.async_copy(stage_v.at[pl.ds(k * 16, 16)],
                             r_out.at[ivp], sem).wait()
            return 0

        lax.fori_loop(0, nk, scat, 0)

    nj = jnp.where(t == 0, jnp.int32(62), jnp.int32(61))

    def giter(j, _):
        g = t + _NW * j
        base = g * _GSZ
        pltpu.async_copy(emb_t.at[:, pl.ds(base, _GSZ)], slab_v, sem).wait()
        process_group(g, base, slab_v, _GSZ)
        return 0

    lax.fori_loop(0, nj, giter, 0)

    tail_base = (n_ent // 128) * 128

    @pl.when(t == (tail_base // _GSZ) % _NW)
    def _tail():
        pltpu.sync_copy(aux, aux_v)
        process_group(jnp.int32(tail_base // _GSZ), jnp.int32(tail_base),
                      aux_v, 128)


def _score_body(nbw, gscale, r_in, out_p, out_n,
                rows_a, rows_b, p_stage, n_stage, sem):
    wid = lax.axis_index("s") * _NC + lax.axis_index("c")
    lane = lax.broadcasted_iota(jnp.int32, (_LANES,), 0)
    base_i = wid * (nbw * _RPB)
    tpc = 8
    cpw = nbw // tpc
    crows = tpc * _RPB

    bufs = [rows_a, rows_b]
    first = pltpu.async_copy(r_in.at[pl.ds(base_i, crows)], bufs[0], sem)

    def compute_chunk(rows_v, chi):
        def row(i, c):
            return rows_v[i, pl.ds(c * _LANES, _LANES)]

        def one_task(tl, carry):
            bo = tl * _RPB
            r = []
            for c in range(4):
                acc = row(bo + _KS, c) - row(bo, c)
                for k in range(1, _KS):
                    acc = acc + (row(bo + _KS + k, c) - row(bo + k, c))
                r.append(acc * (1.0 / _KS))

            g = None
            for j in range(_KS):
                dp = [row(bo + j, c) + r[c] - row(bo + _KS + j, c)
                      for c in range(4)]
                sp = jnp.sum(sum(v * v for v in dp))
                dn = [row(bo + 2 * _KS + j, c) + r[c]
                      - row(bo + 3 * _KS + j, c) for c in range(4)]
                sn = jnp.sum(sum(v * v for v in dn))
                rsp = _rsqrt(sp)
                rsn = _rsqrt(sn)
                ind = (_MARGIN - sn * rsn + sp * rsp) > 0.0
                cp_ = jnp.where(ind, rsp, 0.0)
                cn_ = jnp.where(ind, rsn, 0.0)
                contrib = [cp_ * dp[c] - cn_ * dn[c] for c in range(4)]
                g = contrib if g is None else [g[c] + contrib[c]
                                               for c in range(4)]

            relq = [r[c] - gscale * g[c] for c in range(4)]

            lb = chi * tpc + tl
            qb = bo + 4 * _KS
            pacc = jnp.zeros((_LANES,), jnp.float32)
            for q in range(_KQ):
                dq = [row(qb + q, c) + relq[c] - row(qb + _KQ + q, c)
                      for c in range(4)]
                s = jnp.sum(sum(v * v for v in dq))
                pacc = jnp.where(lane == q, -(s * _rsqrt(s)), pacc)
            p_stage[pl.ds(lb * _KQ, _LANES)] = pacc
            nb = qb + 2 * _KQ
            nacc = jnp.zeros((_LANES,), jnp.float32)
            for q in range(_KQN):
                dq = [row(nb + q, c) + relq[c] - row(nb + _KQN + q, c)
                      for c in range(4)]
                s = jnp.sum(sum(v * v for v in dq))
                nacc = jnp.where(lane == q, -(s * _rsqrt(s)), nacc)
            n_stage[pl.ds(lb * _KQN, _LANES)] = nacc
            return carry

        lax.fori_loop(0, tpc, one_task, 0)

    for chi in range(cpw):
        if chi == 0:
            first.wait()
        nxt = None
        if chi + 1 < cpw:
            nxt = pltpu.async_copy(
                r_in.at[pl.ds(base_i + (chi + 1) * crows, crows)],
                bufs[(chi + 1) % 2], sem)
        compute_chunk(bufs[chi % 2], chi)
        if nxt is not None:
            nxt.wait()

    pltpu.sync_copy(p_stage.at[pl.ds(0, nbw * _KQ)],
                    out_p.at[pl.ds(wid * nbw * _KQ, nbw * _KQ)])
    pltpu.sync_copy(n_stage.at[pl.ds(0, nbw * _KQN)],
                    out_n.at[pl.ds(wid * nbw * _KQN, nbw * _KQN)])


def kernel(embeddings, support_idx, support_neg_idx, query_idx,
           query_neg_idx):
    b = support_idx.shape[0]
    n_ent, d = embeddings.shape
    n_inst = b * _RPB
    nbw = b // _NW
    gscale = _BETA / (b * _KS)

    emb_t = embeddings.T
    tail_base = (n_ent // 128) * 128
    aux = jnp.pad(embeddings[tail_base:].T,
                  ((0, 0), (0, 128 - (n_ent - tail_base))))

    elist = jnp.concatenate([
        support_idx[:, :, 0], support_idx[:, :, 1],
        support_neg_idx[:, :, 0], support_neg_idx[:, :, 1],
        query_idx[:, :, 0], query_idx[:, :, 1],
        query_neg_idx[:, :, 0], query_neg_idx[:, :, 1],
    ], axis=1).reshape(n_inst)

    mesh = plsc.VectorSubcoreMesh(core_axis_name="c", subcore_axis_name="s")
    params = pltpu.CompilerParams(needs_layout_passes=False,
                                  use_tc_tiling_on_sc=True)

    extract = pl.kernel(
        functools.partial(_extract_body, n_inst, n_ent),
        mesh=mesh,
        out_type=jax.ShapeDtypeStruct((n_inst + 128, 128), jnp.float32),
        scratch_types=[
            pltpu.VMEM((n_inst // 16,), jnp.int32),
            pltpu.VMEM((_CAP_LOC + 16,), jnp.int32),
            pltpu.VMEM((_CAP_LOC + 16,), jnp.int32),
            pltpu.VMEM((_CAP_GRP + 16,), jnp.int32),
            pltpu.VMEM((_CAP_GRP + 16,), jnp.int32),
            pltpu.VMEM((d, _GSZ), jnp.float32),
            pltpu.VMEM((d, 128), jnp.float32),
            pltpu.VMEM((_CAP_GRP, 128), jnp.float32),
            pltpu.SemaphoreType.DMA,
        ],
        compiler_params=params,
    )
    rows = extract(emb_t, aux, elist)

    score = pl.kernel(
        functools.partial(_score_body, nbw, gscale),
        mesh=mesh,
        out_type=(
            jax.ShapeDtypeStruct((b * _KQ,), jnp.float32),
            jax.ShapeDtypeStruct((b * _KQN,), jnp.float32),
        ),
        scratch_types=[
            pltpu.VMEM((8 * _RPB, 128), jnp.float32),
            pltpu.VMEM((8 * _RPB, 128), jnp.float32),
            pltpu.VMEM((nbw * _KQ + 8,), jnp.float32),
            pltpu.VMEM((nbw * _KQN + 8,), jnp.float32),
            pltpu.SemaphoreType.DMA,
        ],
        compiler_params=params,
    )
    p_flat, n_flat = score(rows)
    return p_flat.reshape(b, _KQ), n_flat.reshape(b, _KQN)


# speedup vs baseline: 1.6100x; 1.1097x over previous
import functools

import jax
import jax.numpy as jnp
from jax import lax
from jax.experimental import pallas as pl
from jax.experimental.pallas import tpu as pltpu
from jax.experimental.pallas import tpu_sc as plsc

_BETA = 0.05
_MARGIN = 1.0
_LANES = 16

_KS, _KSN, _KQ, _KQN = 5, 5, 10, 10
_RPB = 2 * (_KS + _KSN + _KQ + _KQN)

_NC, _NS = 2, 16
_NW = _NC * _NS

_GSZ = 512
_CAP_LOC = 4096
_CAP_SUB = 768
_CAP_GRP = 128


def _rsqrt(x):
    i = lax.bitcast_convert_type(x, jnp.int32)
    i = jnp.int32(0x5F3759DF) - lax.shift_right_arithmetic(i, jnp.int32(1))
    y = lax.bitcast_convert_type(i, jnp.float32)
    for _ in range(3):
        y = y * (1.5 - 0.5 * x * y * y)
    return y


def _extract_body(n_inst, n_ent, emb_t, aux, elist, r_out,
                  el_v, loc_e, loc_i, sub_e, sub_i,
                  ge_a, gi_a, ge_b, gi_b, slab_a, slab_b, stage_a, stage_b,
                  scnt, sem_sa, sem_sb, sem_ca, sem_cb):
    t = lax.axis_index("s") * _NC + lax.axis_index("c")
    lane = lax.broadcasted_iota(jnp.int32, (_LANES,), 0)
    n_grp = -(-n_ent // _GSZ)
    tail_g = n_grp - 1
    base_cap = ((n_ent - _GSZ) // 128) * 128

    ch = 32
    chw = n_inst // ch

    def chunk_body(ci, off):
        pltpu.sync_copy(elist.at[pl.ds(ci * chw, chw)], el_v)

        def vloop(vi, off):
            for u in range(12):
                vbase = (vi * 12 + u) * 16
                ev = el_v[pl.ds(vbase, 16)]
                m = (lax.shift_right_logical(ev, 9) & (_NW - 1)) == t
                c = plsc.all_reduce_population_count(m)[0]
                plsc.store_compressed(loc_e.at[pl.ds(off, 16)], ev, mask=m)
                iv = ci * chw + vbase + lane
                plsc.store_compressed(loc_i.at[pl.ds(off, 16)], iv, mask=m)
                off = jnp.minimum(off + c, _CAP_LOC)
            return off

        return lax.fori_loop(0, chw // 192, vloop, off)

    cnt_loc = lax.fori_loop(0, ch, chunk_body, jnp.int32(0))

    nv_loc = (cnt_loc + 15) // 16
    for s in range(8):
        def split(v, soff, s=s):
            ev = loc_e[pl.ds(v * 16, 16)]
            iv = loc_i[pl.ds(v * 16, 16)]
            valid = (v * 16 + lane) < cnt_loc
            m = valid & (lax.shift_right_logical(ev, 17) == s)
            c = plsc.all_reduce_population_count(m)[0]
            plsc.store_compressed(sub_e.at[pl.ds(s * _CAP_SUB + soff, 16)],
                                  ev, mask=m)
            plsc.store_compressed(sub_i.at[pl.ds(s * _CAP_SUB + soff, 16)],
                                  iv, mask=m)
            return jnp.minimum(soff + c, _CAP_SUB - 16)

        scnt[s] = lax.fori_loop(0, nv_loc, split, jnp.int32(0))

    def process_group(g, base, src, stage, ge, gi, sem_c, wait_each):
        sg = jnp.clip(lax.shift_right_logical(jnp.maximum(g, 0), 8), 0, 7)
        sbase = sg * _CAP_SUB
        cnt_s = scnt[sg]
        nv = (cnt_s + 15) // 16

        def rescan(v, goff):
            ev = sub_e[pl.ds(sbase + v * 16, 16)]
            iv = sub_i[pl.ds(sbase + v * 16, 16)]
            valid = (v * 16 + lane) < cnt_s
            m = valid & (lax.shift_right_logical(ev, 9) == g)
            c = plsc.all_reduce_population_count(m)[0]
            plsc.store_compressed(ge.at[pl.ds(goff, 16)], ev, mask=m)
            plsc.store_compressed(gi.at[pl.ds(goff, 16)], iv, mask=m)
            return jnp.minimum(goff + c, _CAP_GRP)

        cnt_g = lax.fori_loop(0, nv, rescan, jnp.int32(0))
        nk = (cnt_g + 15) // 16

        def extract(k, _):
            ev = ge[pl.ds(k * 16, 16)]
            for u in range(16):
                col = jnp.clip(ev[u] - base, 0, _GSZ - 1)
                colv = jnp.broadcast_to(col, (_LANES,))
                for c4 in range(4):
                    rowv = lane + c4 * 16
                    val = plsc.load_gather(src, [rowv, colv])
                    stage[k * 16 + u, pl.ds(c4 * 16, 16)] = val
            return 0

        lax.fori_loop(0, nk, extract, 0)

        def scat(k, _):
            iv = gi[pl.ds(k * 16, 16)]
            valid = (k * 16 + lane) < cnt_g
            ivp = jnp.where(valid, iv, n_inst + lane)
            cp = pltpu.async_copy(stage.at[pl.ds(k * 16, 16)],
                                  r_out.at[ivp], sem_c)
            if wait_each:
                cp.wait()
            return 0

        lax.fori_loop(0, nk, scat, 0)
        return nk

    def slab_base(g):
        return jnp.minimum(g * _GSZ, base_cap)

    def fire_slab(g, buf, sem):
        pltpu.async_copy(emb_t.at[:, pl.ds(slab_base(g), _GSZ)], buf, sem)

    def wait_slab(g, buf, sem):
        pltpu.make_async_copy(emb_t.at[:, pl.ds(slab_base(g), _GSZ)],
                              buf, sem).wait()

    def drain_scat(n, stage, sem):
        def d(k, _):
            pltpu.make_async_copy(stage.at[pl.ds(0, 16)],
                                  r_out.at[pl.ds(0, 16)], sem).wait()
            return 0

        lax.fori_loop(0, n, d, 0)

    npair = 31
    fire_slab(t, slab_a, sem_sa)
    fire_slab(t + _NW, slab_b, sem_sb)

    def pair(i, carry):
        nka_p, nkb_p = carry
        ga = t + 2 * _NW * i
        gb = ga + _NW
        gae = jnp.where(ga >= tail_g, jnp.int32(-1), ga)
        gbe = jnp.where(gb >= tail_g, jnp.int32(-1), gb)
        wait_slab(ga, slab_a, sem_sa)
        drain_scat(nka_p, stage_a, sem_ca)
        nka = process_group(gae, slab_base(ga), slab_a, stage_a,
                            ge_a, gi_a, sem_ca, False)
        fire_slab(ga + 2 * _NW, slab_a, sem_sa)
        wait_slab(gb, slab_b, sem_sb)
        drain_scat(nkb_p, stage_b, sem_cb)
        nkb = process_group(gbe, slab_base(gb), slab_b, stage_b,
                            ge_b, gi_b, sem_cb, False)
        fire_slab(gb + 2 * _NW, slab_b, sem_sb)
        return nka, nkb

    nka, nkb = lax.fori_loop(0, npair, pair, (jnp.int32(0), jnp.int32(0)))
    wait_slab(t + 2 * _NW * npair, slab_a, sem_sa)
    wait_slab(t + 2 * _NW * npair + _NW, slab_b, sem_sb)
    drain_scat(nka, stage_a, sem_ca)
    drain_scat(nkb, stage_b, sem_cb)

    tail_base = (n_ent // 128) * 128

    @pl.when(t == tail_g % _NW)
    def _tail():
        pltpu.sync_copy(aux, slab_b.at[:, pl.ds(0, 128)])
        process_group(jnp.int32(tail_g), jnp.int32(tail_base), slab_b,
                      stage_b, ge_b, gi_b, sem_cb, True)


def _score_body(nbw, gscale, r_in, out_p, out_n,
                rows_a, rows_b, p_stage, n_stage, sem):
    wid = lax.axis_index("s") * _NC + lax.axis_index("c")
    lane = lax.broadcasted_iota(jnp.int32, (_LANES,), 0)
    base_i = wid * (nbw * _RPB)
    tpc = 8
    cpw = nbw // tpc
    crows = tpc * _RPB

    bufs = [rows_a, rows_b]
    first = pltpu.async_copy(r_in.at[pl.ds(base_i, crows)], bufs[0], sem)

    def compute_chunk(rows_v, chi):
        def row(i, c):
            return rows_v[i, pl.ds(c * _LANES, _LANES)]

        def one_task(tl, carry):
            bo = tl * _RPB
            r = []
            for c in range(4):
                acc = row(bo + _KS, c) - row(bo, c)
                for k in range(1, _KS):
                    acc = acc + (row(bo + _KS + k, c) - row(bo + k, c))
                r.append(acc * (1.0 / _KS))

            g = None
            for j in range(_KS):
                dp = [row(bo + j, c) + r[c] - row(bo + _KS + j, c)
                      for c in range(4)]
                sp = jnp.sum(sum(v * v for v in dp))
                dn = [row(bo + 2 * _KS + j, c) + r[c]
                      - row(bo + 3 * _KS + j, c) for c in range(4)]
                sn = jnp.sum(sum(v * v for v in dn))
                rsp = _rsqrt(sp)
                rsn = _rsqrt(sn)
                ind = (_MARGIN - sn * rsn + sp * rsp) > 0.0
                cp_ = jnp.where(ind, rsp, 0.0)
                cn_ = jnp.where(ind, rsn, 0.0)
                contrib = [cp_ * dp[c] - cn_ * dn[c] for c in range(4)]
                g = contrib if g is None else [g[c] + contrib[c]
                                               for c in range(4)]

            relq = [r[c] - gscale * g[c] for c in range(4)]

            lb = chi * tpc + tl
            qb = bo + 4 * _KS
            pacc = jnp.zeros((_LANES,), jnp.float32)
            for q in range(_KQ):
                dq = [row(qb + q, c) + relq[c] - row(qb + _KQ + q, c)
                      for c in range(4)]
                s = jnp.sum(sum(v * v for v in dq))
                pacc = jnp.where(lane == q, -(s * _rsqrt(s)), pacc)
            p_stage[pl.ds(lb * _KQ, _LANES)] = pacc
            nb = qb + 2 * _KQ
            nacc = jnp.zeros((_LANES,), jnp.float32)
            for q in range(_KQN):
                dq = [row(nb + q, c) + relq[c] - row(nb + _KQN + q, c)
                      for c in range(4)]
                s = jnp.sum(sum(v * v for v in dq))
                nacc = jnp.where(lane == q, -(s * _rsqrt(s)), nacc)
            n_stage[pl.ds(lb * _KQN, _LANES)] = nacc
            return carry

        lax.fori_loop(0, tpc, one_task, 0)

    for chi in range(cpw):
        if chi == 0:
            first.wait()
        nxt = None
        if chi + 1 < cpw:
            nxt = pltpu.async_copy(
                r_in.at[pl.ds(base_i + (chi + 1) * crows, crows)],
                bufs[(chi + 1) % 2], sem)
        compute_chunk(bufs[chi % 2], chi)
        if nxt is not None:
            nxt.wait()

    pltpu.sync_copy(p_stage.at[pl.ds(0, nbw * _KQ)],
                    out_p.at[pl.ds(wid * nbw * _KQ, nbw * _KQ)])
    pltpu.sync_copy(n_stage.at[pl.ds(0, nbw * _KQN)],
                    out_n.at[pl.ds(wid * nbw * _KQN, nbw * _KQN)])


def kernel(embeddings, support_idx, support_neg_idx, query_idx,
           query_neg_idx):
    b = support_idx.shape[0]
    n_ent, d = embeddings.shape
    n_inst = b * _RPB
    nbw = b // _NW
    gscale = _BETA / (b * _KS)

    emb_t = embeddings.T
    tail_base = (n_ent // 128) * 128
    aux = jnp.pad(embeddings[tail_base:].T,
                  ((0, 0), (0, 128 - (n_ent - tail_base))))

    elist = jnp.concatenate([
        support_idx[:, :, 0], support_idx[:, :, 1],
        support_neg_idx[:, :, 0], support_neg_idx[:, :, 1],
        query_idx[:, :, 0], query_idx[:, :, 1],
        query_neg_idx[:, :, 0], query_neg_idx[:, :, 1],
    ], axis=1).reshape(n_inst)

    mesh = plsc.VectorSubcoreMesh(core_axis_name="c", subcore_axis_name="s")
    params = pltpu.CompilerParams(needs_layout_passes=False,
                                  use_tc_tiling_on_sc=True)

    extract = pl.kernel(
        functools.partial(_extract_body, n_inst, n_ent),
        mesh=mesh,
        out_type=jax.ShapeDtypeStruct((n_inst + 128, 128), jnp.float32),
        scratch_types=[
            pltpu.VMEM((n_inst // 32,), jnp.int32),
            pltpu.VMEM((_CAP_LOC + 16,), jnp.int32),
            pltpu.VMEM((_CAP_LOC + 16,), jnp.int32),
            pltpu.VMEM((8 * _CAP_SUB,), jnp.int32),
            pltpu.VMEM((8 * _CAP_SUB,), jnp.int32),
            pltpu.VMEM((_CAP_GRP + 16,), jnp.int32),
            pltpu.VMEM((_CAP_GRP + 16,), jnp.int32),
            pltpu.VMEM((_CAP_GRP + 16,), jnp.int32),
            pltpu.VMEM((_CAP_GRP + 16,), jnp.int32),
            pltpu.VMEM((d, _GSZ), jnp.float32),
            pltpu.VMEM((d, _GSZ), jnp.float32),
            pltpu.VMEM((_CAP_GRP, 128), jnp.float32),
            pltpu.VMEM((_CAP_GRP, 128), jnp.float32),
            pltpu.SMEM((8,), jnp.int32),
            pltpu.SemaphoreType.DMA,
            pltpu.SemaphoreType.DMA,
            pltpu.SemaphoreType.DMA,
            pltpu.SemaphoreType.DMA,
        ],
        compiler_params=params,
    )
    rows = extract(emb_t, aux, elist)

    score = pl.kernel(
        functools.partial(_score_body, nbw, gscale),
        mesh=mesh,
        out_type=(
            jax.ShapeDtypeStruct((b * _KQ,), jnp.float32),
            jax.ShapeDtypeStruct((b * _KQN,), jnp.float32),
        ),
        scratch_types=[
            pltpu.VMEM((8 * _RPB, 128), jnp.float32),
            pltpu.VMEM((8 * _RPB, 128), jnp.float32),
            pltpu.VMEM((nbw * _KQ + 8,), jnp.float32),
            pltpu.VMEM((nbw * _KQN + 8,), jnp.float32),
            pltpu.SemaphoreType.DMA,
        ],
        compiler_params=params,
    )
    p_flat, n_flat = score(rows)
    return p_flat.reshape(b, _KQ), n_flat.reshape(b, _KQN)


# speedup vs baseline: 1.6145x; 1.0028x over previous
import functools

import jax
import jax.numpy as jnp
from jax import lax
from jax.experimental import pallas as pl
from jax.experimental.pallas import tpu as pltpu
from jax.experimental.pallas import tpu_sc as plsc

_BETA = 0.05
_MARGIN = 1.0
_LANES = 16

_KS, _KSN, _KQ, _KQN = 5, 5, 10, 10
_RPB = 2 * (_KS + _KSN + _KQ + _KQN)

_NC, _NS = 2, 16
_NW = _NC * _NS

_GSZ = 512
_CAP_LOC = 4096
_CAP_SUB = 768
_CAP_GRP = 128


def _rsqrt(x):
    i = lax.bitcast_convert_type(x, jnp.int32)
    i = jnp.int32(0x5F3759DF) - lax.shift_right_arithmetic(i, jnp.int32(1))
    y = lax.bitcast_convert_type(i, jnp.float32)
    for _ in range(3):
        y = y * (1.5 - 0.5 * x * y * y)
    return y


def _extract_body(n_inst, n_ent, emb_t, aux, elist, r_out,
                  el_v, loc_e, loc_i, sub_e, sub_i,
                  ge_a, gi_a, ge_b, gi_b, slab_a, slab_b, stage_a, stage_b,
                  scnt, sem_sa, sem_sb, sem_ca, sem_cb):
    t = lax.axis_index("s") * _NC + lax.axis_index("c")
    lane = lax.broadcasted_iota(jnp.int32, (_LANES,), 0)
    n_grp = -(-n_ent // _GSZ)
    tail_g = n_grp - 1
    base_cap = ((n_ent - _GSZ) // 128) * 128

    ch = 32
    chw = n_inst // ch

    def chunk_body(ci, off):
        pltpu.sync_copy(elist.at[pl.ds(ci * chw, chw)], el_v)

        def vloop(vi, off):
            for u in range(12):
                vbase = (vi * 12 + u) * 16
                ev = el_v[pl.ds(vbase, 16)]
                m = (lax.shift_right_logical(ev, 9) & (_NW - 1)) == t
                c = plsc.all_reduce_population_count(m)[0]
                plsc.store_compressed(loc_e.at[pl.ds(off, 16)], ev, mask=m)
                iv = ci * chw + vbase + lane
                plsc.store_compressed(loc_i.at[pl.ds(off, 16)], iv, mask=m)
                off = jnp.minimum(off + c, _CAP_LOC)
            return off

        return lax.fori_loop(0, chw // 192, vloop, off)

    cnt_loc = lax.fori_loop(0, ch, chunk_body, jnp.int32(0))

    nv_loc = (cnt_loc + 15) // 16
    for s in range(8):
        def split(v, soff, s=s):
            ev = loc_e[pl.ds(v * 16, 16)]
            iv = loc_i[pl.ds(v * 16, 16)]
            valid = (v * 16 + lane) < cnt_loc
            m = valid & (lax.shift_right_logical(ev, 17) == s)
            c = plsc.all_reduce_population_count(m)[0]
            plsc.store_compressed(sub_e.at[pl.ds(s * _CAP_SUB + soff, 16)],
                                  ev, mask=m)
            plsc.store_compressed(sub_i.at[pl.ds(s * _CAP_SUB + soff, 16)],
                                  iv, mask=m)
            return jnp.minimum(soff + c, _CAP_SUB - 16)

        scnt[s] = lax.fori_loop(0, nv_loc, split, jnp.int32(0))

    def process_group(g, base, src, stage, ge, gi, sem_c, wait_each):
        sg = jnp.clip(lax.shift_right_logical(jnp.maximum(g, 0), 8), 0, 7)
        sbase = sg * _CAP_SUB
        cnt_s = scnt[sg]
        nv = (cnt_s + 15) // 16

        def rescan(v, goff):
            ev = sub_e[pl.ds(sbase + v * 16, 16)]
            iv = sub_i[pl.ds(sbase + v * 16, 16)]
            valid = (v * 16 + lane) < cnt_s
            m = valid & (lax.shift_right_logical(ev, 9) == g)
            c = plsc.all_reduce_population_count(m)[0]
            plsc.store_compressed(ge.at[pl.ds(goff, 16)], ev, mask=m)
            plsc.store_compressed(gi.at[pl.ds(goff, 16)], iv, mask=m)
            return jnp.minimum(goff + c, _CAP_GRP)

        cnt_g = lax.fori_loop(0, nv, rescan, jnp.int32(0))
        nk = (cnt_g + 15) // 16

        def extract(k, _):
            ev = ge[pl.ds(k * 16, 16)]
            cols = jnp.clip(ev - base, 0, _GSZ - 1)
            for u in range(16):
                uu = jnp.full((_LANES, 1), u, jnp.int32)
                colv = lax.gather(
                    cols, uu,
                    lax.GatherDimensionNumbers(offset_dims=(),
                                               collapsed_slice_dims=(0,),
                                               start_index_map=(0,)),
                    (1,), mode=lax.GatherScatterMode.PROMISE_IN_BOUNDS)
                for c4 in range(4):
                    rowv = lane + c4 * 16
                    val = plsc.load_gather(src, [rowv, colv])
                    stage[k * 16 + u, pl.ds(c4 * 16, 16)] = val
            return 0

        lax.fori_loop(0, nk, extract, 0)

        def scat(k, _):
            iv = gi[pl.ds(k * 16, 16)]
            valid = (k * 16 + lane) < cnt_g
            ivp = jnp.where(valid, iv, n_inst + lane)
            cp = pltpu.async_copy(stage.at[pl.ds(k * 16, 16)],
                                  r_out.at[ivp], sem_c)
            if wait_each:
                cp.wait()
            return 0

        lax.fori_loop(0, nk, scat, 0)
        return nk

    def slab_base(g):
        return jnp.minimum(g * _GSZ, base_cap)

    def fire_slab(g, buf, sem):
        pltpu.async_copy(emb_t.at[:, pl.ds(slab_base(g), _GSZ)], buf, sem)

    def wait_slab(g, buf, sem):
        pltpu.make_async_copy(emb_t.at[:, pl.ds(slab_base(g), _GSZ)],
                              buf, sem).wait()

    def drain_scat(n, stage, sem):
        def d(k, _):
            pltpu.make_async_copy(stage.at[pl.ds(0, 16)],
                                  r_out.at[pl.ds(0, 16)], sem).wait()
            return 0

        lax.fori_loop(0, n, d, 0)

    npair = 31
    fire_slab(t, slab_a, sem_sa)
    fire_slab(t + _NW, slab_b, sem_sb)

    def pair(i, carry):
        nka_p, nkb_p = carry
        ga = t + 2 * _NW * i
        gb = ga + _NW
        gae = jnp.where(ga >= tail_g, jnp.int32(-1), ga)
        gbe = jnp.where(gb >= tail_g, jnp.int32(-1), gb)
        wait_slab(ga, slab_a, sem_sa)
        drain_scat(nka_p, stage_a, sem_ca)
        nka = process_group(gae, slab_base(ga), slab_a, stage_a,
                            ge_a, gi_a, sem_ca, False)
        fire_slab(ga + 2 * _NW, slab_a, sem_sa)
        wait_slab(gb, slab_b, sem_sb)
        drain_scat(nkb_p, stage_b, sem_cb)
        nkb = process_group(gbe, slab_base(gb), slab_b, stage_b,
                            ge_b, gi_b, sem_cb, False)
        fire_slab(gb + 2 * _NW, slab_b, sem_sb)
        return nka, nkb

    nka, nkb = lax.fori_loop(0, npair, pair, (jnp.int32(0), jnp.int32(0)))
    wait_slab(t + 2 * _NW * npair, slab_a, sem_sa)
    wait_slab(t + 2 * _NW * npair + _NW, slab_b, sem_sb)
    drain_scat(nka, stage_a, sem_ca)
    drain_scat(nkb, stage_b, sem_cb)

    tail_base = (n_ent // 128) * 128

    @pl.when(t == tail_g % _NW)
    def _tail():
        pltpu.sync_copy(aux, slab_b.at[:, pl.ds(0, 128)])
        process_group(jnp.int32(tail_g), jnp.int32(tail_base), slab_b,
                      stage_b, ge_b, gi_b, sem_cb, True)


def _score_body(nbw, gscale, r_in, out_p, out_n,
                rows_a, rows_b, p_stage, n_stage, sem):
    wid = lax.axis_index("s") * _NC + lax.axis_index("c")
    lane = lax.broadcasted_iota(jnp.int32, (_LANES,), 0)
    base_i = wid * (nbw * _RPB)
    tpc = 8
    cpw = nbw // tpc
    crows = tpc * _RPB

    bufs = [rows_a, rows_b]
    first = pltpu.async_copy(r_in.at[pl.ds(base_i, crows)], bufs[0], sem)

    def compute_chunk(rows_v, chi):
        def row(i, c):
            return rows_v[i, pl.ds(c * _LANES, _LANES)]

        def one_task(tl, carry):
            bo = tl * _RPB
            r = []
            for c in range(4):
                acc = row(bo + _KS, c) - row(bo, c)
                for k in range(1, _KS):
                    acc = acc + (row(bo + _KS + k, c) - row(bo + k, c))
                r.append(acc * (1.0 / _KS))

            g = None
            for j in range(_KS):
                dp = [row(bo + j, c) + r[c] - row(bo + _KS + j, c)
                      for c in range(4)]
                sp = jnp.sum(sum(v * v for v in dp))
                dn = [row(bo + 2 * _KS + j, c) + r[c]
                      - row(bo + 3 * _KS + j, c) for c in range(4)]
                sn = jnp.sum(sum(v * v for v in dn))
                rsp = _rsqrt(sp)
                rsn = _rsqrt(sn)
                ind = (_MARGIN - sn * rsn + sp * rsp) > 0.0
                cp_ = jnp.where(ind, rsp, 0.0)
                cn_ = jnp.where(ind, rsn, 0.0)
                contrib = [cp_ * dp[c] - cn_ * dn[c] for c in range(4)]
                g = contrib if g is None else [g[c] + contrib[c]
                                               for c in range(4)]

            relq = [r[c] - gscale * g[c] for c in range(4)]

            lb = chi * tpc + tl
            qb = bo + 4 * _KS
            pacc = jnp.zeros((_LANES,), jnp.float32)
            for q in range(_KQ):
                dq = [row(qb + q, c) + relq[c] - row(qb + _KQ + q, c)
                      for c in range(4)]
                s = jnp.sum(sum(v * v for v in dq))
                pacc = jnp.where(lane == q, -(s * _rsqrt(s)), pacc)
            p_stage[pl.ds(lb * _KQ, _LANES)] = pacc
            nb = qb + 2 * _KQ
            nacc = jnp.zeros((_LANES,), jnp.float32)
            for q in range(_KQN):
                dq = [row(nb + q, c) + relq[c] - row(nb + _KQN + q, c)
                      for c in range(4)]
                s = jnp.sum(sum(v * v for v in dq))
                nacc = jnp.where(lane == q, -(s * _rsqrt(s)), nacc)
            n_stage[pl.ds(lb * _KQN, _LANES)] = nacc
            return carry

        lax.fori_loop(0, tpc, one_task, 0)

    for chi in range(cpw):
        if chi == 0:
            first.wait()
        nxt = None
        if chi + 1 < cpw:
            nxt = pltpu.async_copy(
                r_in.at[pl.ds(base_i + (chi + 1) * crows, crows)],
                bufs[(chi + 1) % 2], sem)
        compute_chunk(bufs[chi % 2], chi)
        if nxt is not None:
            nxt.wait()

    pltpu.sync_copy(p_stage.at[pl.ds(0, nbw * _KQ)],
                    out_p.at[pl.ds(wid * nbw * _KQ, nbw * _KQ)])
    pltpu.sync_copy(n_stage.at[pl.ds(0, nbw * _KQN)],
                    out_n.at[pl.ds(wid * nbw * _KQN, nbw * _KQN)])


def kernel(embeddings, support_idx, support_neg_idx, query_idx,
           query_neg_idx):
    b = support_idx.shape[0]
    n_ent, d = embeddings.shape
    n_inst = b * _RPB
    nbw = b // _NW
    gscale = _BETA / (b * _KS)

    emb_t = embeddings.T
    tail_base = (n_ent // 128) * 128
    aux = jnp.pad(embeddings[tail_base:].T,
                  ((0, 0), (0, 128 - (n_ent - tail_base))))

    elist = jnp.concatenate([
        support_idx[:, :, 0], support_idx[:, :, 1],
        support_neg_idx[:, :, 0], support_neg_idx[:, :, 1],
        query_idx[:, :, 0], query_idx[:, :, 1],
        query_neg_idx[:, :, 0], query_neg_idx[:, :, 1],
    ], axis=1).reshape(n_inst)

    mesh = plsc.VectorSubcoreMesh(core_axis_name="c", subcore_axis_name="s")
    params = pltpu.CompilerParams(needs_layout_passes=False,
                                  use_tc_tiling_on_sc=True)

    extract = pl.kernel(
        functools.partial(_extract_body, n_inst, n_ent),
        mesh=mesh,
        out_type=jax.ShapeDtypeStruct((n_inst + 128, 128), jnp.float32),
        scratch_types=[
            pltpu.VMEM((n_inst // 32,), jnp.int32),
            pltpu.VMEM((_CAP_LOC + 16,), jnp.int32),
            pltpu.VMEM((_CAP_LOC + 16,), jnp.int32),
            pltpu.VMEM((8 * _CAP_SUB,), jnp.int32),
            pltpu.VMEM((8 * _CAP_SUB,), jnp.int32),
            pltpu.VMEM((_CAP_GRP + 16,), jnp.int32),
            pltpu.VMEM((_CAP_GRP + 16,), jnp.int32),
            pltpu.VMEM((_CAP_GRP + 16,), jnp.int32),
            pltpu.VMEM((_CAP_GRP + 16,), jnp.int32),
            pltpu.VMEM((d, _GSZ), jnp.float32),
            pltpu.VMEM((d, _GSZ), jnp.float32),
            pltpu.VMEM((_CAP_GRP, 128), jnp.float32),
            pltpu.VMEM((_CAP_GRP, 128), jnp.float32),
            pltpu.SMEM((8,), jnp.int32),
            pltpu.SemaphoreType.DMA,
            pltpu.SemaphoreType.DMA,
            pltpu.SemaphoreType.DMA,
            pltpu.SemaphoreType.DMA,
        ],
        compiler_params=params,
    )
    rows = extract(emb_t, aux, elist)

    score = pl.kernel(
        functools.partial(_score_body, nbw, gscale),
        mesh=mesh,
        out_type=(
            jax.ShapeDtypeStruct((b * _KQ,), jnp.float32),
            jax.ShapeDtypeStruct((b * _KQN,), jnp.float32),
        ),
        scratch_types=[
            pltpu.VMEM((8 * _RPB, 128), jnp.float32),
            pltpu.VMEM((8 * _RPB, 128), jnp.float32),
            pltpu.VMEM((nbw * _KQ + 8,), jnp.float32),
            pltpu.VMEM((nbw * _KQN + 8,), jnp.float32),
            pltpu.SemaphoreType.DMA,
        ],
        compiler_params=params,
    )
    p_flat, n_flat = score(rows)
    return p_flat.reshape(b, _KQ), n_flat.reshape(b, _KQN)


# speedup vs baseline: 1.6500x; 1.0220x over previous
import functools

import jax
import jax.numpy as jnp
from jax import lax
from jax.experimental import pallas as pl
from jax.experimental.pallas import tpu as pltpu
from jax.experimental.pallas import tpu_sc as plsc

_BETA = 0.05
_MARGIN = 1.0
_LANES = 16

_KS, _KSN, _KQ, _KQN = 5, 5, 10, 10
_RPB = 2 * (_KS + _KSN + _KQ + _KQN)

_NC, _NS = 2, 16
_NW = _NC * _NS

_GSZ = 512
_CAP_SEG = 1536
_CAP_SUB = 768
_CAP_GRP = 128


def _rsqrt(x):
    i = lax.bitcast_convert_type(x, jnp.int32)
    i = jnp.int32(0x5F3759DF) - lax.shift_right_arithmetic(i, jnp.int32(1))
    y = lax.bitcast_convert_type(i, jnp.float32)
    for _ in range(3):
        y = y * (1.5 - 0.5 * x * y * y)
    return y


def _extract_body(n_inst, n_ent, emb_t, aux, elist, r_out,
                  el_v, loc_e, loc_i, sub_e, sub_i,
                  ge_a, gi_a, ge_b, gi_b, slab_a, slab_b, stage_a, stage_b,
                  scnt, sem_sa, sem_sb, sem_ca, sem_cb):
    t = lax.axis_index("s") * _NC + lax.axis_index("c")
    lane = lax.broadcasted_iota(jnp.int32, (_LANES,), 0)
    n_grp = -(-n_ent // _GSZ)
    tail_g = n_grp - 1
    base_cap = ((n_ent - _GSZ) // 128) * 128

    pltpu.async_copy(emb_t.at[:, pl.ds(t * _GSZ, _GSZ)], slab_a, sem_sa)
    pltpu.async_copy(emb_t.at[:, pl.ds((t + _NW) * _GSZ, _GSZ)],
                     slab_b, sem_sb)

    ch = 32
    chw = n_inst // ch

    def chunk_body(ci, offs):
        pltpu.sync_copy(elist.at[pl.ds(ci * chw, chw)], el_v)

        def vloop(vi, offs):
            ol = list(offs)
            for u in range(12):
                cn = u % 4
                vbase = (vi * 12 + u) * 16
                ev = el_v[pl.ds(vbase, 16)]
                m = (lax.shift_right_logical(ev, 9) & (_NW - 1)) == t
                c = plsc.all_reduce_population_count(m)[0]
                seg = cn * _CAP_SEG
                plsc.store_compressed(loc_e.at[pl.ds(seg + ol[cn], 16)],
                                      ev, mask=m)
                iv = ci * chw + vbase + lane
                plsc.store_compressed(loc_i.at[pl.ds(seg + ol[cn], 16)],
                                      iv, mask=m)
                ol[cn] = jnp.minimum(ol[cn] + c, _CAP_SEG - 16)
            return tuple(ol)

        return lax.fori_loop(0, chw // 192, vloop, offs)

    z = jnp.int32(0)
    cnt_segs = lax.fori_loop(0, ch, chunk_body, (z, z, z, z))

    for s in range(8):
        soff = jnp.int32(0)
        for seg in range(4):
            cseg = cnt_segs[seg]
            nvs = (cseg + 15) // 16

            def split(v, soff, s=s, seg=seg, cseg=cseg):
                ev = loc_e[pl.ds(seg * _CAP_SEG + v * 16, 16)]
                iv = loc_i[pl.ds(seg * _CAP_SEG + v * 16, 16)]
                valid = (v * 16 + lane) < cseg
                m = valid & (lax.shift_right_logical(ev, 17) == s)
                c = plsc.all_reduce_population_count(m)[0]
                plsc.store_compressed(
                    sub_e.at[pl.ds(s * _CAP_SUB + soff, 16)], ev, mask=m)
                plsc.store_compressed(
                    sub_i.at[pl.ds(s * _CAP_SUB + soff, 16)], iv, mask=m)
                return jnp.minimum(soff + c, _CAP_SUB - 16)

            soff = lax.fori_loop(0, nvs, split, soff)
        scnt[s] = soff

    def process_group(g, base, src, stage, ge, gi, sem_c, wait_each):
        sg = jnp.clip(lax.shift_right_logical(jnp.maximum(g, 0), 8), 0, 7)
        sbase = sg * _CAP_SUB
        cnt_s = scnt[sg]
        nv = (cnt_s + 15) // 16

        def rescan(v, goff):
            ev = sub_e[pl.ds(sbase + v * 16, 16)]
            iv = sub_i[pl.ds(sbase + v * 16, 16)]
            valid = (v * 16 + lane) < cnt_s
            m = valid & (lax.shift_right_logical(ev, 9) == g)
            c = plsc.all_reduce_population_count(m)[0]
            plsc.store_compressed(ge.at[pl.ds(goff, 16)], ev, mask=m)
            plsc.store_compressed(gi.at[pl.ds(goff, 16)], iv, mask=m)
            return jnp.minimum(goff + c, _CAP_GRP)

        cnt_g = lax.fori_loop(0, nv, rescan, jnp.int32(0))
        nk = (cnt_g + 15) // 16

        def extract(k, _):
            ev = ge[pl.ds(k * 16, 16)]
            cols = jnp.clip(ev - base, 0, _GSZ - 1)
            for u in range(16):
                uu = jnp.full((_LANES, 1), u, jnp.int32)
                colv = lax.gather(
                    cols, uu,
                    lax.GatherDimensionNumbers(offset_dims=(),
                                               collapsed_slice_dims=(0,),
                                               start_index_map=(0,)),
                    (1,), mode=lax.GatherScatterMode.PROMISE_IN_BOUNDS)
                for c4 in range(4):
                    rowv = lane + c4 * 16
                    val = plsc.load_gather(src, [rowv, colv])
                    stage[k * 16 + u, pl.ds(c4 * 16, 16)] = val
            return 0

        lax.fori_loop(0, nk, extract, 0)

        def scat(k, _):
            iv = gi[pl.ds(k * 16, 16)]
            valid = (k * 16 + lane) < cnt_g
            ivp = jnp.where(valid, iv, n_inst + lane)
            cp = pltpu.async_copy(stage.at[pl.ds(k * 16, 16)],
                                  r_out.at[ivp], sem_c)
            if wait_each:
                cp.wait()
            return 0

        lax.fori_loop(0, nk, scat, 0)
        return nk

    def slab_base(g):
        return jnp.minimum(g * _GSZ, base_cap)

    def fire_slab(g, buf, sem):
        pltpu.async_copy(emb_t.at[:, pl.ds(slab_base(g), _GSZ)], buf, sem)

    def wait_slab(g, buf, sem):
        pltpu.make_async_copy(emb_t.at[:, pl.ds(slab_base(g), _GSZ)],
                              buf, sem).wait()

    def drain_scat(n, stage, sem):
        def d(k, _):
            pltpu.make_async_copy(stage.at[pl.ds(0, 16)],
                                  r_out.at[pl.ds(0, 16)], sem).wait()
            return 0

        lax.fori_loop(0, n, d, 0)

    npair = 31

    def pair(i, carry):
        nka_p, nkb_p = carry
        ga = t + 2 * _NW * i
        gb = ga + _NW
        gae = jnp.where(ga >= tail_g, jnp.int32(-1), ga)
        gbe = jnp.where(gb >= tail_g, jnp.int32(-1), gb)
        wait_slab(ga, slab_a, sem_sa)
        drain_scat(nka_p, stage_a, sem_ca)
        nka = process_group(gae, slab_base(ga), slab_a, stage_a,
                            ge_a, gi_a, sem_ca, False)
        fire_slab(ga + 2 * _NW, slab_a, sem_sa)
        wait_slab(gb, slab_b, sem_sb)
        drain_scat(nkb_p, stage_b, sem_cb)
        nkb = process_group(gbe, slab_base(gb), slab_b, stage_b,
                            ge_b, gi_b, sem_cb, False)
        fire_slab(gb + 2 * _NW, slab_b, sem_sb)
        return nka, nkb

    nka, nkb = lax.fori_loop(0, npair, pair, (jnp.int32(0), jnp.int32(0)))
    wait_slab(t + 2 * _NW * npair, slab_a, sem_sa)
    wait_slab(t + 2 * _NW * npair + _NW, slab_b, sem_sb)
    drain_scat(nka, stage_a, sem_ca)
    drain_scat(nkb, stage_b, sem_cb)

    tail_base = (n_ent // 128) * 128

    @pl.when(t == tail_g % _NW)
    def _tail():
        pltpu.sync_copy(aux, slab_b.at[:, pl.ds(0, 128)])
        process_group(jnp.int32(tail_g), jnp.int32(tail_base), slab_b,
                      stage_b, ge_b, gi_b, sem_cb, True)


def _score_body(nbw, gscale, r_in, out_p, out_n,
                rows_a, rows_b, p_stage, n_stage, sem):
    wid = lax.axis_index("s") * _NC + lax.axis_index("c")
    lane = lax.broadcasted_iota(jnp.int32, (_LANES,), 0)
    base_i = wid * (nbw * _RPB)
    tpc = 8
    cpw = nbw // tpc
    crows = tpc * _RPB

    bufs = [rows_a, rows_b]
    first = pltpu.async_copy(r_in.at[pl.ds(base_i, crows)], bufs[0], sem)

    def compute_chunk(rows_v, chi):
        def row(i, c):
            return rows_v[i, pl.ds(c * _LANES, _LANES)]

        def one_task(tl, carry):
            bo = tl * _RPB
            r = []
            for c in range(4):
                acc = row(bo + _KS, c) - row(bo, c)
                for k in range(1, _KS):
                    acc = acc + (row(bo + _KS + k, c) - row(bo + k, c))
                r.append(acc * (1.0 / _KS))

            g = None
            for j in range(_KS):
                dp = [row(bo + j, c) + r[c] - row(bo + _KS + j, c)
                      for c in range(4)]
                sp = jnp.sum(sum(v * v for v in dp))
                dn = [row(bo + 2 * _KS + j, c) + r[c]
                      - row(bo + 3 * _KS + j, c) for c in range(4)]
                sn = jnp.sum(sum(v * v for v in dn))
                rsp = _rsqrt(sp)
                rsn = _rsqrt(sn)
                ind = (_MARGIN - sn * rsn + sp * rsp) > 0.0
                cp_ = jnp.where(ind, rsp, 0.0)
                cn_ = jnp.where(ind, rsn, 0.0)
                contrib = [cp_ * dp[c] - cn_ * dn[c] for c in range(4)]
                g = contrib if g is None else [g[c] + contrib[c]
                                               for c in range(4)]

            relq = [r[c] - gscale * g[c] for c in range(4)]

            lb = chi * tpc + tl
            qb = bo + 4 * _KS
            pacc = jnp.zeros((_LANES,), jnp.float32)
            for q in range(_KQ):
                dq = [row(qb + q, c) + relq[c] - row(qb + _KQ + q, c)
                      for c in range(4)]
                s = jnp.sum(sum(v * v for v in dq))
                pacc = jnp.where(lane == q, -(s * _rsqrt(s)), pacc)
            p_stage[pl.ds(lb * _KQ, _LANES)] = pacc
            nb = qb + 2 * _KQ
            nacc = jnp.zeros((_LANES,), jnp.float32)
            for q in range(_KQN):
                dq = [row(nb + q, c) + relq[c] - row(nb + _KQN + q, c)
                      for c in range(4)]
                s = jnp.sum(sum(v * v for v in dq))
                nacc = jnp.where(lane == q, -(s * _rsqrt(s)), nacc)
            n_stage[pl.ds(lb * _KQN, _LANES)] = nacc
            return carry

        lax.fori_loop(0, tpc, one_task, 0)

    for chi in range(cpw):
        if chi == 0:
            first.wait()
        nxt = None
        if chi + 1 < cpw:
            nxt = pltpu.async_copy(
                r_in.at[pl.ds(base_i + (chi + 1) * crows, crows)],
                bufs[(chi + 1) % 2], sem)
        compute_chunk(bufs[chi % 2], chi)
        if nxt is not None:
            nxt.wait()

    pltpu.sync_copy(p_stage.at[pl.ds(0, nbw * _KQ)],
                    out_p.at[pl.ds(wid * nbw * _KQ, nbw * _KQ)])
    pltpu.sync_copy(n_stage.at[pl.ds(0, nbw * _KQN)],
                    out_n.at[pl.ds(wid * nbw * _KQN, nbw * _KQN)])


def kernel(embeddings, support_idx, support_neg_idx, query_idx,
           query_neg_idx):
    b = support_idx.shape[0]
    n_ent, d = embeddings.shape
    n_inst = b * _RPB
    nbw = b // _NW
    gscale = _BETA / (b * _KS)

    emb_t = embeddings.T
    tail_base = (n_ent // 128) * 128
    aux = jnp.pad(embeddings[tail_base:].T,
                  ((0, 0), (0, 128 - (n_ent - tail_base))))

    elist = jnp.concatenate([
        support_idx[:, :, 0], support_idx[:, :, 1],
        support_neg_idx[:, :, 0], support_neg_idx[:, :, 1],
        query_idx[:, :, 0], query_idx[:, :, 1],
        query_neg_idx[:, :, 0], query_neg_idx[:, :, 1],
    ], axis=1).reshape(n_inst)

    mesh = plsc.VectorSubcoreMesh(core_axis_name="c", subcore_axis_name="s")
    params = pltpu.CompilerParams(needs_layout_passes=False,
                                  use_tc_tiling_on_sc=True)

    extract = pl.kernel(
        functools.partial(_extract_body, n_inst, n_ent),
        mesh=mesh,
        out_type=jax.ShapeDtypeStruct((n_inst + 128, 128), jnp.float32),
        scratch_types=[
            pltpu.VMEM((n_inst // 32,), jnp.int32),
            pltpu.VMEM((4 * _CAP_SEG + 16,), jnp.int32),
            pltpu.VMEM((4 * _CAP_SEG + 16,), jnp.int32),
            pltpu.VMEM((8 * _CAP_SUB,), jnp.int32),
            pltpu.VMEM((8 * _CAP_SUB,), jnp.int32),
            pltpu.VMEM((_CAP_GRP + 16,), jnp.int32),
            pltpu.VMEM((_CAP_GRP + 16,), jnp.int32),
            pltpu.VMEM((_CAP_GRP + 16,), jnp.int32),
            pltpu.VMEM((_CAP_GRP + 16,), jnp.int32),
            pltpu.VMEM((d, _GSZ), jnp.float32),
            pltpu.VMEM((d, _GSZ), jnp.float32),
            pltpu.VMEM((_CAP_GRP, 128), jnp.float32),
            pltpu.VMEM((_CAP_GRP, 128), jnp.float32),
            pltpu.SMEM((8,), jnp.int32),
            pltpu.SemaphoreType.DMA,
            pltpu.SemaphoreType.DMA,
            pltpu.SemaphoreType.DMA,
            pltpu.SemaphoreType.DMA,
        ],
        compiler_params=params,
    )
    rows = extract(emb_t, aux, elist)

    score = pl.kernel(
        functools.partial(_score_body, nbw, gscale),
        mesh=mesh,
        out_type=(
            jax.ShapeDtypeStruct((b * _KQ,), jnp.float32),
            jax.ShapeDtypeStruct((b * _KQN,), jnp.float32),
        ),
        scratch_types=[
            pltpu.VMEM((8 * _RPB, 128), jnp.float32),
            pltpu.VMEM((8 * _RPB, 128), jnp.float32),
            pltpu.VMEM((nbw * _KQ + 8,), jnp.float32),
            pltpu.VMEM((nbw * _KQN + 8,), jnp.float32),
            pltpu.SemaphoreType.DMA,
        ],
        compiler_params=params,
    )
    p_flat, n_flat = score(rows)
    return p_flat.reshape(b, _KQ), n_flat.reshape(b, _KQN)
